# baseline (speedup 1.0000x reference)
import functools

import jax
import jax.numpy as jnp
from jax import lax
from jax.experimental import pallas as pl
from jax.experimental.pallas import tpu as pltpu
from jax.experimental.pallas import tpu_sc as plsc

DIM = 64
L = 16
NC, NS = 2, 16
NW = NC * NS
CHUNK = 128


def _cosine_body(batch, oid_hbm, eid_hbm, m2_hbm, e2_hbm, out_hbm,
                 oid_v, eid_v, opair_v, epair_v, orows_v, erows_v, out_v, sem):
    bpw = batch // NW
    nchunk = bpw // CHUNK
    wid = lax.axis_index("s") * NC + lax.axis_index("c")
    cbase = wid * nchunk

    pltpu.sync_copy(oid_hbm.at[pl.ds(cbase, nchunk)], oid_v)
    pltpu.sync_copy(eid_hbm.at[pl.ds(cbase, nchunk)], eid_v)

    for c in range(nchunk):
        for v in range(CHUNK // L):
            opair_v[c, pl.ds(v * L, L)] = lax.shift_right_logical(
                oid_v[c, pl.ds(v * L, L)], 1)
            epair_v[c, pl.ds(v * L, L)] = lax.shift_right_logical(
                eid_v[c, pl.ds(v * L, L)], 1)

    def gather(c, slot):
        return (pltpu.async_copy(m2_hbm.at[opair_v.at[c]], orows_v.at[slot], sem),
                pltpu.async_copy(e2_hbm.at[epair_v.at[c]], erows_v.at[slot], sem))

    lane = lax.iota(jnp.int32, L)

    def compute(c, slot):
        def group(j, _):
            iv_o = oid_v[c, pl.ds(j * L, L)]
            iv_e = eid_v[c, pl.ds(j * L, L)]
            dotv = jnp.zeros((L,), jnp.float32)
            pv = jnp.zeros((L,), jnp.float32)
            for r in range(L):
                row = j * L + r
                ho = (iv_o[r] & 1) * DIM
                he = (iv_e[r] & 1) * DIM
                dot = jnp.zeros((L,), jnp.float32)
                on = jnp.zeros((L,), jnp.float32)
                en = jnp.zeros((L,), jnp.float32)
                for k in range(DIM // L):
                    o = orows_v[slot, row, pl.ds(ho + k * L, L)]
                    e = erows_v[slot, row, pl.ds(he + k * L, L)]
                    dot = dot + o * e
                    on = on + o * o
                    en = en + e * e
                sdot = jnp.sum(dot)
                sp = jnp.sum(on) * jnp.sum(en)
                dotv = jnp.where(lane == r, sdot, dotv)
                pv = jnp.where(lane == r, sp, pv)
            yi = jnp.int32(0x5F3759DF) - lax.shift_right_logical(
                plsc.bitcast(pv, jnp.int32), 1)
            y = plsc.bitcast(yi, jnp.float32)
            xh = pv * jnp.float32(0.5)
            for _ in range(3):
                y = y * (jnp.float32(1.5) - xh * y * y)
            out_v[c, pl.ds(j * L, L)] = dotv * y
            return 0

        lax.fori_loop(0, CHUNK // L, group, 0)

    inflight = gather(0, 0)
    for c in range(nchunk):
        nxt = gather(c + 1, (c + 1) % 2) if c + 1 < nchunk else ()
        for cp in inflight:
            cp.wait()
        compute(c, c % 2)
        inflight = nxt

    pltpu.sync_copy(out_v, out_hbm.at[pl.ds(cbase, nchunk)])


def kernel(o_id, e_id, mentors, mentees):
    batch = o_id.shape[0]
    bpw = batch // NW
    nchunk = bpw // CHUNK
    nv, dim = mentors.shape
    m2 = mentors.reshape(nv // 2, 2 * dim)
    e2 = mentees.reshape(nv // 2, 2 * dim)
    oid2 = o_id.reshape(batch // CHUNK, CHUNK)
    eid2 = e_id.reshape(batch // CHUNK, CHUNK)

    mesh = plsc.VectorSubcoreMesh(core_axis_name="c", subcore_axis_name="s",
                                  num_cores=NC, num_subcores=NS)
    call = pl.kernel(
        functools.partial(_cosine_body, batch),
        out_type=jax.ShapeDtypeStruct((batch // CHUNK, CHUNK), jnp.float32),
        mesh=mesh,
        compiler_params=pltpu.CompilerParams(needs_layout_passes=False,
                                             use_tc_tiling_on_sc=True),
        scratch_types=[
            pltpu.VMEM((nchunk, CHUNK), jnp.int32),
            pltpu.VMEM((nchunk, CHUNK), jnp.int32),
            pltpu.VMEM((nchunk, CHUNK), jnp.int32),
            pltpu.VMEM((nchunk, CHUNK), jnp.int32),
            pltpu.VMEM((2, CHUNK, 2 * DIM), jnp.float32),
            pltpu.VMEM((2, CHUNK, 2 * DIM), jnp.float32),
            pltpu.VMEM((nchunk, CHUNK), jnp.float32),
            pltpu.SemaphoreType.DMA,
        ],
    )
    out2 = call(oid2, eid2, m2, e2)
    return out2.reshape(batch)
